# baseline (speedup 1.0000x reference)
import functools
import math

import jax
import jax.numpy as jnp
from jax import lax
from jax.experimental import pallas as pl
from jax.experimental.pallas import tpu as pltpu
from jax.experimental.pallas import tpu_sc as plsc

_D_A = 768
_D_B = 768
_D_K = 128
_R = 2
_TOP_K = 8
_TBLK = 256
_PBLK = 512
_VBLK = 1024


def _pos_enc(seq_len, d_model):
    pos = jnp.arange(seq_len)[:, None]
    i = jnp.arange(d_model // 2)[None, :]
    angle = pos / 10000 ** (2 * i / d_model)
    enc = jnp.concatenate([jnp.sin(angle), jnp.cos(angle)], axis=-1)
    return enc[:, :d_model]


def _embed_gather(embed_table, ids):
    info = plsc.get_sparse_core_info()
    nw = info.num_cores * info.num_subcores
    t_tot = ids.shape[0]
    d = embed_table.shape[1]
    b_per_w = t_tot // nw
    mesh = plsc.VectorSubcoreMesh(core_axis_name="c", subcore_axis_name="s")

    @functools.partial(
        pl.kernel,
        mesh=mesh,
        out_type=jax.ShapeDtypeStruct((t_tot, d), jnp.float32),
        scratch_types=[
            pltpu.VMEM((b_per_w,), jnp.int32),
            pltpu.VMEM((b_per_w, d), jnp.float32),
            pltpu.SemaphoreType.DMA,
        ],
    )
    def k(table_hbm, idx_hbm, out_hbm, idx_v, rows_v, sem):
        wid = lax.axis_index("s") * info.num_cores + lax.axis_index("c")
        base = wid * b_per_w
        pltpu.sync_copy(idx_hbm.at[pl.ds(base, b_per_w)], idx_v)
        pltpu.async_copy(table_hbm.at[idx_v], rows_v, sem).wait()
        pltpu.sync_copy(rows_v, out_hbm.at[pl.ds(base, b_per_w)])

    return k(embed_table, ids)


def _parta_body(x_ref, pos_ref, wa1_ref, ba1_ref, wa2_ref, ba2_ref,
                qp_ref, wk_ref, ha_ref, kq_ref, zsum):
    i = pl.program_id(0)
    xp = x_ref[...] + pos_ref[...]
    h1 = jax.nn.gelu(
        jnp.dot(xp, wa1_ref[...], preferred_element_type=jnp.float32)
        + ba1_ref[...])
    ha = (jnp.dot(h1, wa2_ref[...], preferred_element_type=jnp.float32)
          + ba2_ref[...] + xp)
    ha_ref[...] = ha

    @pl.when(i == 0)
    def _():
        zsum[...] = jnp.zeros_like(zsum)

    zsum[...] += jnp.sum(ha, axis=0, keepdims=True)

    @pl.when(i == pl.num_programs(0) - 1)
    def _():
        t_tot = pl.num_programs(0) * x_ref.shape[0]
        z = zsum[...] * (1.0 / t_tot)
        q = jnp.dot(z, qp_ref[...], preferred_element_type=jnp.float32)
        kq = lax.dot_general(q, wk_ref[...], (((1,), (1,)), ((), ())),
                             preferred_element_type=jnp.float32)
        kq_ref[...] = kq * (1.0 / math.sqrt(_D_K))


def _scan_body(lam_ref, warm_ref, kq_ref, pool_ref, idx_ref, alpha_ref,
               scores):
    j = pl.program_id(0)
    s = lax.dot_general(kq_ref[...], pool_ref[...], (((1,), (1,)), ((), ())),
                        preferred_element_type=jnp.float32)
    scores[pl.ds(j, 1), :] = s

    @pl.when(j == pl.num_programs(0) - 1)
    def _():
        sc = scores[...]
        row = lax.broadcasted_iota(jnp.int32, sc.shape, 0)
        col = lax.broadcasted_iota(jnp.int32, sc.shape, 1)
        flat = row * sc.shape[1] + col
        big = jnp.int32(2**30)
        neg = jnp.float32(-3.0e38)
        vrows = []
        for t in range(_TOP_K):
            m = jnp.max(sc)
            sel = jnp.min(jnp.where(sc == m, flat, big))
            idx_ref[pl.ds(t, 1), :] = jnp.full((1, 128), sel, jnp.int32)
            vrows.append(jnp.full((1, 128), m, jnp.float32))
            sc = jnp.where(flat == sel, neg, sc)
        vmat = jnp.concatenate(vrows, axis=0)
        tv = vmat * lam_ref[0, 0]
        mx = jnp.max(tv, axis=0, keepdims=True)
        e = jnp.exp(tv - mx)
        sm = e / jnp.sum(e, axis=0, keepdims=True)
        alpha_ref[...] = jnp.where(warm_ref[0, 0] > 0,
                                   jnp.float32(1.0 / _TOP_K), sm)


def _assemble_body(idx_ref, gamma_ref, alpha_ref, wbase_ref, pool_ref,
                   w_ref, gath, sems):
    for k in range(_TOP_K):
        pltpu.make_async_copy(
            pool_ref.at[pl.ds(idx_ref[0, k], 1), :],
            gath.at[pl.ds(k, 1), :], sems.at[k]).start()
    for k in range(_TOP_K):
        pltpu.make_async_copy(
            pool_ref.at[pl.ds(idx_ref[0, k], 1), :],
            gath.at[pl.ds(k, 1), :], sems.at[k]).wait()
    g = gath[...]
    g1 = g[:, : _D_B * _R] * alpha_ref[:, 0:1]
    i_ = lax.broadcasted_iota(jnp.int32, (_D_B * _R, _D_B), 0)
    d_ = lax.broadcasted_iota(jnp.int32, (_D_B * _R, _D_B), 1)
    delta = jnp.zeros((_D_B, _D_A), jnp.float32)
    for r in range(_R):
        e_r = (i_ == _R * d_ + r).astype(jnp.float32)
        u_t = lax.dot_general(e_r, g1, (((0,), (1,)), ((), ())),
                              preferred_element_type=jnp.float32)
        v_r = g[:, _D_B * _R + _D_A * r: _D_B * _R + _D_A * (r + 1)]
        delta += jnp.dot(u_t, v_r, preferred_element_type=jnp.float32)
    w_ref[...] = wbase_ref[...] + gamma_ref[0, 0] * delta


def _mid_partb_body(ha_ref, w_ref, bbase_ref, lns_ref, lnb_ref,
                    wb1_ref, bb1_ref, wb2_ref, bb2_ref, hout_ref):
    ha = ha_ref[...]
    hm = (lax.dot_general(ha, w_ref[...], (((1,), (1,)), ((), ())),
                          preferred_element_type=jnp.float32)
          + bbase_ref[...])
    mu = jnp.mean(hm, axis=1, keepdims=True)
    var = jnp.mean((hm - mu) ** 2, axis=1, keepdims=True)
    hm = (hm - mu) * lax.rsqrt(var + 1e-6) * lns_ref[...] + lnb_ref[...]
    g1 = jax.nn.gelu(
        jnp.dot(hm, wb1_ref[...], preferred_element_type=jnp.float32)
        + bb1_ref[...])
    hout_ref[...] = (
        jnp.dot(g1, wb2_ref[...], preferred_element_type=jnp.float32)
        + bb2_ref[...] + hm)


def _logits_body(hout_ref, wlm_ref, out_ref):
    out_ref[...] = jnp.dot(hout_ref[...], wlm_ref[...],
                           preferred_element_type=jnp.float32)


def kernel(input_ids, lambda_val, is_warmup, embed_table, Wa1, ba1, Wa2, ba2,
           pool_vectors, W_key, q_proj, W_base, b_base, gamma,
           ln_scale, ln_bias, Wb1, bb1, Wb2, bb2, W_lm):
    b, t_tot = input_ids.shape
    vocab, d_a = embed_table.shape
    n_pool, d_pool = pool_vectors.shape

    ids = input_ids.reshape(t_tot).astype(jnp.int32)
    x = _embed_gather(embed_table, ids)
    pos = _pos_enc(t_tot, d_a)

    full = lambda i: (0, 0)
    ha, kq = pl.pallas_call(
        _parta_body,
        grid=(t_tot // _TBLK,),
        in_specs=[
            pl.BlockSpec((_TBLK, d_a), lambda i: (i, 0)),
            pl.BlockSpec((_TBLK, d_a), lambda i: (i, 0)),
            pl.BlockSpec((d_a, d_a), full),
            pl.BlockSpec((1, d_a), full),
            pl.BlockSpec((d_a, d_a), full),
            pl.BlockSpec((1, d_a), full),
            pl.BlockSpec((d_a, _D_K), full),
            pl.BlockSpec((d_pool, _D_K), full),
        ],
        out_specs=[
            pl.BlockSpec((_TBLK, d_a), lambda i: (i, 0)),
            pl.BlockSpec((1, d_pool), full),
        ],
        out_shape=[
            jax.ShapeDtypeStruct((t_tot, d_a), jnp.float32),
            jax.ShapeDtypeStruct((1, d_pool), jnp.float32),
        ],
        scratch_shapes=[pltpu.VMEM((1, d_a), jnp.float32)],
    )(x, pos.astype(jnp.float32), Wa1, ba1.reshape(1, -1), Wa2,
      ba2.reshape(1, -1), q_proj, W_key)

    lam = jnp.asarray(lambda_val, jnp.float32).reshape(1, 1)
    warm = jnp.where(is_warmup, 1.0, 0.0).astype(jnp.float32).reshape(1, 1)
    idxmat, alphamat = pl.pallas_call(
        _scan_body,
        grid=(n_pool // _PBLK,),
        in_specs=[
            pl.BlockSpec(memory_space=pltpu.SMEM),
            pl.BlockSpec(memory_space=pltpu.SMEM),
            pl.BlockSpec((1, d_pool), full),
            pl.BlockSpec((_PBLK, d_pool), lambda j: (j, 0)),
        ],
        out_specs=[
            pl.BlockSpec((_TOP_K, 128), full),
            pl.BlockSpec((_TOP_K, 128), full),
        ],
        out_shape=[
            jax.ShapeDtypeStruct((_TOP_K, 128), jnp.int32),
            jax.ShapeDtypeStruct((_TOP_K, 128), jnp.float32),
        ],
        scratch_shapes=[pltpu.VMEM((n_pool // _PBLK, _PBLK), jnp.float32)],
    )(lam, warm, kq, pool_vectors)

    idx8 = idxmat[:, 0].reshape(1, _TOP_K)
    gam = jnp.asarray(gamma, jnp.float32).reshape(1, 1)
    w2d = pl.pallas_call(
        _assemble_body,
        in_specs=[
            pl.BlockSpec(memory_space=pltpu.SMEM),
            pl.BlockSpec(memory_space=pltpu.SMEM),
            pl.BlockSpec((_TOP_K, 128), lambda: (0, 0)),
            pl.BlockSpec((_D_B, d_a), lambda: (0, 0)),
            pl.BlockSpec(memory_space=pl.ANY),
        ],
        out_specs=pl.BlockSpec((_D_B, d_a), lambda: (0, 0)),
        out_shape=jax.ShapeDtypeStruct((_D_B, d_a), jnp.float32),
        scratch_shapes=[
            pltpu.VMEM((_TOP_K, d_pool), jnp.float32),
            pltpu.SemaphoreType.DMA((_TOP_K,)),
        ],
    )(idx8, gam, alphamat, W_base, pool_vectors)

    hout = pl.pallas_call(
        _mid_partb_body,
        grid=(t_tot // _TBLK,),
        in_specs=[
            pl.BlockSpec((_TBLK, d_a), lambda i: (i, 0)),
            pl.BlockSpec((_D_B, d_a), full),
            pl.BlockSpec((1, _D_B), full),
            pl.BlockSpec((1, _D_B), full),
            pl.BlockSpec((1, _D_B), full),
            pl.BlockSpec((_D_B, _D_B), full),
            pl.BlockSpec((1, _D_B), full),
            pl.BlockSpec((_D_B, _D_B), full),
            pl.BlockSpec((1, _D_B), full),
        ],
        out_specs=pl.BlockSpec((_TBLK, _D_B), lambda i: (i, 0)),
        out_shape=jax.ShapeDtypeStruct((t_tot, _D_B), jnp.float32),
    )(ha, w2d, b_base.reshape(1, -1), ln_scale.reshape(1, -1),
      ln_bias.reshape(1, -1), Wb1, bb1.reshape(1, -1), Wb2,
      bb2.reshape(1, -1))

    logits = pl.pallas_call(
        _logits_body,
        grid=(vocab // _VBLK,),
        in_specs=[
            pl.BlockSpec((t_tot, _D_B), full),
            pl.BlockSpec((_D_B, _VBLK), lambda v: (0, v)),
        ],
        out_specs=pl.BlockSpec((t_tot, _VBLK), lambda v: (0, v)),
        out_shape=jax.ShapeDtypeStruct((t_tot, vocab), jnp.float32),
    )(hout.astype(jnp.bfloat16), W_lm.astype(jnp.bfloat16))

    return (logits.reshape(b, t_tot, vocab),
            alphamat[:, 0].reshape(b, _TOP_K),
            idxmat[:, 0].reshape(b, _TOP_K),
            w2d.reshape(b, _D_B, d_a))


# speedup vs baseline: 1.1736x; 1.1736x over previous
import functools
import math

import jax
import jax.numpy as jnp
from jax import lax
from jax.experimental import pallas as pl
from jax.experimental.pallas import tpu as pltpu
from jax.experimental.pallas import tpu_sc as plsc

_D_A = 768
_D_B = 768
_D_K = 128
_R = 2
_TOP_K = 8
_TBLK = 256
_PBLK = 512
_VBLK = 1024


def _pos_enc(seq_len, d_model):
    pos = jnp.arange(seq_len)[:, None]
    i = jnp.arange(d_model // 2)[None, :]
    angle = pos / 10000 ** (2 * i / d_model)
    enc = jnp.concatenate([jnp.sin(angle), jnp.cos(angle)], axis=-1)
    return enc[:, :d_model]


def _embed_gather(embed_table, ids):
    info = plsc.get_sparse_core_info()
    nw = info.num_cores * info.num_subcores
    t_tot = ids.shape[0]
    d = embed_table.shape[1]
    b_per_w = t_tot // nw
    mesh = plsc.VectorSubcoreMesh(core_axis_name="c", subcore_axis_name="s")

    @functools.partial(
        pl.kernel,
        mesh=mesh,
        out_type=jax.ShapeDtypeStruct((t_tot, d), jnp.float32),
        scratch_types=[
            pltpu.VMEM((b_per_w,), jnp.int32),
            pltpu.VMEM((b_per_w, d), jnp.float32),
            pltpu.SemaphoreType.DMA,
        ],
    )
    def k(table_hbm, idx_hbm, out_hbm, idx_v, rows_v, sem):
        wid = lax.axis_index("s") * info.num_cores + lax.axis_index("c")
        base = wid * b_per_w
        pltpu.sync_copy(idx_hbm.at[pl.ds(base, b_per_w)], idx_v)
        pltpu.async_copy(table_hbm.at[idx_v], rows_v, sem).wait()
        pltpu.sync_copy(rows_v, out_hbm.at[pl.ds(base, b_per_w)])

    return k(embed_table, ids)


def _parta_body(x_ref, pos_ref, wa1_ref, ba1_ref, wa2_ref, ba2_ref,
                qp_ref, wk_ref, ha_ref, kq_ref, zsum):
    i = pl.program_id(0)
    xp = x_ref[...] + pos_ref[...]
    h1 = jax.nn.gelu(
        jnp.dot(xp, wa1_ref[...], preferred_element_type=jnp.float32)
        + ba1_ref[...])
    ha = (jnp.dot(h1, wa2_ref[...], preferred_element_type=jnp.float32)
          + ba2_ref[...] + xp)
    ha_ref[...] = ha

    @pl.when(i == 0)
    def _():
        zsum[...] = jnp.zeros_like(zsum)

    zsum[...] += jnp.sum(ha, axis=0, keepdims=True)

    @pl.when(i == pl.num_programs(0) - 1)
    def _():
        t_tot = pl.num_programs(0) * x_ref.shape[0]
        z = zsum[...] * (1.0 / t_tot)
        q = jnp.dot(z, qp_ref[...], preferred_element_type=jnp.float32)
        kq = lax.dot_general(q, wk_ref[...], (((1,), (1,)), ((), ())),
                             preferred_element_type=jnp.float32)
        kq_ref[...] = kq * (1.0 / math.sqrt(_D_K))


def _scan_body(lam_ref, warm_ref, kq_ref, pool_ref, idx_ref, alpha_ref,
               scores):
    j = pl.program_id(0)
    s = lax.dot_general(kq_ref[...], pool_ref[...], (((1,), (1,)), ((), ())),
                        preferred_element_type=jnp.float32)
    scores[pl.ds(j, 1), :] = s

    @pl.when(j == pl.num_programs(0) - 1)
    def _():
        sc = scores[...]
        row = lax.broadcasted_iota(jnp.int32, sc.shape, 0)
        col = lax.broadcasted_iota(jnp.int32, sc.shape, 1)
        flat = row * sc.shape[1] + col
        big = jnp.int32(2**30)
        neg = jnp.float32(-3.0e38)
        vrows = []
        for t in range(_TOP_K):
            m = jnp.max(sc)
            sel = jnp.min(jnp.where(sc == m, flat, big))
            idx_ref[pl.ds(t, 1), :] = jnp.full((1, 128), sel, jnp.int32)
            vrows.append(jnp.full((1, 128), m, jnp.float32))
            sc = jnp.where(flat == sel, neg, sc)
        vmat = jnp.concatenate(vrows, axis=0)
        tv = vmat * lam_ref[0, 0]
        mx = jnp.max(tv, axis=0, keepdims=True)
        e = jnp.exp(tv - mx)
        sm = e / jnp.sum(e, axis=0, keepdims=True)
        alpha_ref[...] = jnp.where(warm_ref[0, 0] > 0,
                                   jnp.float32(1.0 / _TOP_K), sm)


def _final_body(idx_ref, gamma_ref, alpha_ref, wbase_ref, pool_ref, ha_ref,
                bbase_ref, lns_ref, lnb_ref, wb1_ref, bb1_ref, wb2_ref,
                bb2_ref, wlm_ref, out_ref, w_ref, gath, sems, hout_sc):
    v = pl.program_id(0)

    @pl.when(v == 0)
    def _():
        for k in range(_TOP_K):
            pltpu.make_async_copy(
                pool_ref.at[pl.ds(idx_ref[0, k], 1), :],
                gath.at[pl.ds(k, 1), :], sems.at[k]).start()
        for k in range(_TOP_K):
            pltpu.make_async_copy(
                pool_ref.at[pl.ds(idx_ref[0, k], 1), :],
                gath.at[pl.ds(k, 1), :], sems.at[k]).wait()
        g = gath[...]
        g1 = g[:, : _D_B * _R] * alpha_ref[:, 0:1]
        i_ = lax.broadcasted_iota(jnp.int32, (_D_B * _R, _D_B), 0)
        d_ = lax.broadcasted_iota(jnp.int32, (_D_B * _R, _D_B), 1)
        delta = jnp.zeros((_D_B, _D_A), jnp.float32)
        for r in range(_R):
            e_r = (i_ == _R * d_ + r).astype(jnp.float32)
            u_t = lax.dot_general(e_r, g1, (((0,), (1,)), ((), ())),
                                  preferred_element_type=jnp.float32)
            v_r = g[:, _D_B * _R + _D_A * r: _D_B * _R + _D_A * (r + 1)]
            delta += jnp.dot(u_t, v_r, preferred_element_type=jnp.float32)
        w = wbase_ref[...] + gamma_ref[0, 0] * delta
        w_ref[...] = w
        ha = ha_ref[...].astype(jnp.bfloat16)
        hm = (lax.dot_general(ha, w.astype(jnp.bfloat16),
                              (((1,), (1,)), ((), ())),
                              preferred_element_type=jnp.float32)
              + bbase_ref[...])
        mu = jnp.mean(hm, axis=1, keepdims=True)
        var = jnp.mean((hm - mu) ** 2, axis=1, keepdims=True)
        hm = (hm - mu) * lax.rsqrt(var + 1e-6) * lns_ref[...] + lnb_ref[...]
        g1b = jax.nn.gelu(
            jnp.dot(hm.astype(jnp.bfloat16), wb1_ref[...].astype(jnp.bfloat16),
                    preferred_element_type=jnp.float32)
            + bb1_ref[...])
        hout = (jnp.dot(g1b.astype(jnp.bfloat16),
                        wb2_ref[...].astype(jnp.bfloat16),
                        preferred_element_type=jnp.float32)
                + bb2_ref[...] + hm)
        hout_sc[...] = hout.astype(jnp.bfloat16)

    out_ref[...] = jnp.dot(hout_sc[...], wlm_ref[...].astype(jnp.bfloat16),
                           preferred_element_type=jnp.float32)


def kernel(input_ids, lambda_val, is_warmup, embed_table, Wa1, ba1, Wa2, ba2,
           pool_vectors, W_key, q_proj, W_base, b_base, gamma,
           ln_scale, ln_bias, Wb1, bb1, Wb2, bb2, W_lm):
    b, t_tot = input_ids.shape
    vocab, d_a = embed_table.shape
    n_pool, d_pool = pool_vectors.shape

    ids = input_ids.reshape(t_tot).astype(jnp.int32)
    x = _embed_gather(embed_table, ids)
    pos = _pos_enc(t_tot, d_a)

    full = lambda i: (0, 0)
    ha, kq = pl.pallas_call(
        _parta_body,
        grid=(t_tot // _TBLK,),
        in_specs=[
            pl.BlockSpec((_TBLK, d_a), lambda i: (i, 0)),
            pl.BlockSpec((_TBLK, d_a), lambda i: (i, 0)),
            pl.BlockSpec((d_a, d_a), full),
            pl.BlockSpec((1, d_a), full),
            pl.BlockSpec((d_a, d_a), full),
            pl.BlockSpec((1, d_a), full),
            pl.BlockSpec((d_a, _D_K), full),
            pl.BlockSpec((d_pool, _D_K), full),
        ],
        out_specs=[
            pl.BlockSpec((_TBLK, d_a), lambda i: (i, 0)),
            pl.BlockSpec((1, d_pool), full),
        ],
        out_shape=[
            jax.ShapeDtypeStruct((t_tot, d_a), jnp.float32),
            jax.ShapeDtypeStruct((1, d_pool), jnp.float32),
        ],
        scratch_shapes=[pltpu.VMEM((1, d_a), jnp.float32)],
    )(x, pos.astype(jnp.float32), Wa1, ba1.reshape(1, -1), Wa2,
      ba2.reshape(1, -1), q_proj, W_key)

    lam = jnp.asarray(lambda_val, jnp.float32).reshape(1, 1)
    warm = jnp.where(is_warmup, 1.0, 0.0).astype(jnp.float32).reshape(1, 1)
    idxmat, alphamat = pl.pallas_call(
        _scan_body,
        grid=(n_pool // _PBLK,),
        in_specs=[
            pl.BlockSpec(memory_space=pltpu.SMEM),
            pl.BlockSpec(memory_space=pltpu.SMEM),
            pl.BlockSpec((1, d_pool), full),
            pl.BlockSpec((_PBLK, d_pool), lambda j: (j, 0)),
        ],
        out_specs=[
            pl.BlockSpec((_TOP_K, 128), full),
            pl.BlockSpec((_TOP_K, 128), full),
        ],
        out_shape=[
            jax.ShapeDtypeStruct((_TOP_K, 128), jnp.int32),
            jax.ShapeDtypeStruct((_TOP_K, 128), jnp.float32),
        ],
        scratch_shapes=[pltpu.VMEM((n_pool // _PBLK, _PBLK), jnp.float32)],
    )(lam, warm, kq, pool_vectors)

    idx8 = idxmat[:, 0].reshape(1, _TOP_K)
    gam = jnp.asarray(gamma, jnp.float32).reshape(1, 1)
    logits, w2d = pl.pallas_call(
        _final_body,
        grid=(vocab // _VBLK,),
        in_specs=[
            pl.BlockSpec(memory_space=pltpu.SMEM),
            pl.BlockSpec(memory_space=pltpu.SMEM),
            pl.BlockSpec((_TOP_K, 128), full),
            pl.BlockSpec((_D_B, d_a), full),
            pl.BlockSpec(memory_space=pl.ANY),
            pl.BlockSpec((t_tot, d_a), full),
            pl.BlockSpec((1, _D_B), full),
            pl.BlockSpec((1, _D_B), full),
            pl.BlockSpec((1, _D_B), full),
            pl.BlockSpec((_D_B, _D_B), full),
            pl.BlockSpec((1, _D_B), full),
            pl.BlockSpec((_D_B, _D_B), full),
            pl.BlockSpec((1, _D_B), full),
            pl.BlockSpec((_D_B, _VBLK), lambda v: (0, v)),
        ],
        out_specs=[
            pl.BlockSpec((t_tot, _VBLK), lambda v: (0, v)),
            pl.BlockSpec((_D_B, d_a), full),
        ],
        out_shape=[
            jax.ShapeDtypeStruct((t_tot, vocab), jnp.float32),
            jax.ShapeDtypeStruct((_D_B, d_a), jnp.float32),
        ],
        scratch_shapes=[
            pltpu.VMEM((_TOP_K, d_pool), jnp.float32),
            pltpu.SemaphoreType.DMA((_TOP_K,)),
            pltpu.VMEM((t_tot, _D_B), jnp.bfloat16),
        ],
    )(idx8, gam, alphamat, W_base, pool_vectors, ha, b_base.reshape(1, -1),
      ln_scale.reshape(1, -1), ln_bias.reshape(1, -1), Wb1,
      bb1.reshape(1, -1), Wb2, bb2.reshape(1, -1), W_lm)

    return (logits.reshape(b, t_tot, vocab),
            alphamat[:, 0].reshape(b, _TOP_K),
            idxmat[:, 0].reshape(b, _TOP_K),
            w2d.reshape(b, _D_B, d_a))


# speedup vs baseline: 1.1893x; 1.0134x over previous
import functools
import math

import jax
import jax.numpy as jnp
from jax import lax
from jax.experimental import pallas as pl
from jax.experimental.pallas import tpu as pltpu
from jax.experimental.pallas import tpu_sc as plsc

_D_A = 768
_D_B = 768
_D_K = 128
_R = 2
_TOP_K = 8
_TBLK = 256
_PBLK = 1024
_VBLK = 1024


def _pos_enc(seq_len, d_model):
    pos = jnp.arange(seq_len)[:, None]
    i = jnp.arange(d_model // 2)[None, :]
    angle = pos / 10000 ** (2 * i / d_model)
    enc = jnp.concatenate([jnp.sin(angle), jnp.cos(angle)], axis=-1)
    return enc[:, :d_model]


def _embed_gather(embed_table, ids):
    info = plsc.get_sparse_core_info()
    nw = info.num_cores * info.num_subcores
    t_tot = ids.shape[0]
    d = embed_table.shape[1]
    b_per_w = t_tot // nw
    mesh = plsc.VectorSubcoreMesh(core_axis_name="c", subcore_axis_name="s")

    @functools.partial(
        pl.kernel,
        mesh=mesh,
        out_type=jax.ShapeDtypeStruct((t_tot, d), jnp.float32),
        scratch_types=[
            pltpu.VMEM((b_per_w,), jnp.int32),
            pltpu.VMEM((b_per_w, d), jnp.float32),
            pltpu.SemaphoreType.DMA,
        ],
    )
    def k(table_hbm, idx_hbm, out_hbm, idx_v, rows_v, sem):
        wid = lax.axis_index("s") * info.num_cores + lax.axis_index("c")
        base = wid * b_per_w
        pltpu.sync_copy(idx_hbm.at[pl.ds(base, b_per_w)], idx_v)
        pltpu.async_copy(table_hbm.at[idx_v], rows_v, sem).wait()
        pltpu.sync_copy(rows_v, out_hbm.at[pl.ds(base, b_per_w)])

    return k(embed_table, ids)


def _parta_body(x_ref, pos_ref, wa1_ref, ba1_ref, wa2_ref, ba2_ref,
                qp_ref, wk_ref, ha_ref, kq_ref, zsum):
    i = pl.program_id(0)
    xp = x_ref[...] + pos_ref[...]
    h1 = jax.nn.gelu(
        jnp.dot(xp, wa1_ref[...], preferred_element_type=jnp.float32)
        + ba1_ref[...])
    ha = (jnp.dot(h1, wa2_ref[...], preferred_element_type=jnp.float32)
          + ba2_ref[...] + xp)
    ha_ref[...] = ha

    @pl.when(i == 0)
    def _():
        zsum[...] = jnp.zeros_like(zsum)

    zsum[...] += jnp.sum(ha, axis=0, keepdims=True)

    @pl.when(i == pl.num_programs(0) - 1)
    def _():
        t_tot = pl.num_programs(0) * x_ref.shape[0]
        z = zsum[...] * (1.0 / t_tot)
        q = jnp.dot(z, qp_ref[...], preferred_element_type=jnp.float32)
        kq = lax.dot_general(q, wk_ref[...], (((1,), (1,)), ((), ())),
                             preferred_element_type=jnp.float32)
        kq_ref[...] = kq * (1.0 / math.sqrt(_D_K))


def _scan_body(lam_ref, warm_ref, kq_ref, pool_ref, idx_ref, alpha_ref,
               scores):
    j = pl.program_id(0)
    s = lax.dot_general(kq_ref[...], pool_ref[...], (((1,), (1,)), ((), ())),
                        preferred_element_type=jnp.float32)
    scores[pl.ds(j, 1), :] = s

    @pl.when(j == pl.num_programs(0) - 1)
    def _():
        sc = scores[...]
        row = lax.broadcasted_iota(jnp.int32, sc.shape, 0)
        col = lax.broadcasted_iota(jnp.int32, sc.shape, 1)
        flat = row * sc.shape[1] + col
        big = jnp.int32(2**30)
        neg = jnp.float32(-3.0e38)
        vrows = []
        for t in range(_TOP_K):
            m = jnp.max(sc)
            sel = jnp.min(jnp.where(sc == m, flat, big))
            idx_ref[pl.ds(t, 1), :] = jnp.full((1, 128), sel, jnp.int32)
            vrows.append(jnp.full((1, 128), m, jnp.float32))
            sc = jnp.where(flat == sel, neg, sc)
        vmat = jnp.concatenate(vrows, axis=0)
        tv = vmat * lam_ref[0, 0]
        mx = jnp.max(tv, axis=0, keepdims=True)
        e = jnp.exp(tv - mx)
        sm = e / jnp.sum(e, axis=0, keepdims=True)
        alpha_ref[...] = jnp.where(warm_ref[0, 0] > 0,
                                   jnp.float32(1.0 / _TOP_K), sm)


def _final_body(idx_ref, gamma_ref, alpha_ref, wbase_ref, pool_ref, ha_ref,
                bbase_ref, lns_ref, lnb_ref, wb1_ref, bb1_ref, wb2_ref,
                bb2_ref, wlm_ref, out_ref, w_ref, gath, sems, hout_sc):
    v = pl.program_id(0)

    @pl.when(v == 0)
    def _():
        for k in range(_TOP_K):
            pltpu.make_async_copy(
                pool_ref.at[pl.ds(idx_ref[0, k], 1), :],
                gath.at[pl.ds(k, 1), :], sems.at[k]).start()
        for k in range(_TOP_K):
            pltpu.make_async_copy(
                pool_ref.at[pl.ds(idx_ref[0, k], 1), :],
                gath.at[pl.ds(k, 1), :], sems.at[k]).wait()
        g = gath[...]
        g1 = g[:, : _D_B * _R] * alpha_ref[:, 0:1]
        i_ = lax.broadcasted_iota(jnp.int32, (_D_B * _R, _D_B), 0)
        d_ = lax.broadcasted_iota(jnp.int32, (_D_B * _R, _D_B), 1)
        delta = jnp.zeros((_D_B, _D_A), jnp.float32)
        for r in range(_R):
            e_r = (i_ == _R * d_ + r).astype(jnp.float32)
            u_t = lax.dot_general(e_r, g1, (((0,), (1,)), ((), ())),
                                  preferred_element_type=jnp.float32)
            v_r = g[:, _D_B * _R + _D_A * r: _D_B * _R + _D_A * (r + 1)]
            delta += jnp.dot(u_t, v_r, preferred_element_type=jnp.float32)
        w = wbase_ref[...] + gamma_ref[0, 0] * delta
        w_ref[...] = w
        ha = ha_ref[...].astype(jnp.bfloat16)
        hm = (lax.dot_general(ha, w.astype(jnp.bfloat16),
                              (((1,), (1,)), ((), ())),
                              preferred_element_type=jnp.float32)
              + bbase_ref[...])
        mu = jnp.mean(hm, axis=1, keepdims=True)
        var = jnp.mean((hm - mu) ** 2, axis=1, keepdims=True)
        hm = (hm - mu) * lax.rsqrt(var + 1e-6) * lns_ref[...] + lnb_ref[...]
        g1b = jax.nn.gelu(
            jnp.dot(hm.astype(jnp.bfloat16), wb1_ref[...].astype(jnp.bfloat16),
                    preferred_element_type=jnp.float32)
            + bb1_ref[...])
        hout = (jnp.dot(g1b.astype(jnp.bfloat16),
                        wb2_ref[...].astype(jnp.bfloat16),
                        preferred_element_type=jnp.float32)
                + bb2_ref[...] + hm)
        hout_sc[...] = hout.astype(jnp.bfloat16)

    out_ref[...] = jnp.dot(hout_sc[...], wlm_ref[...].astype(jnp.bfloat16),
                           preferred_element_type=jnp.float32)


def kernel(input_ids, lambda_val, is_warmup, embed_table, Wa1, ba1, Wa2, ba2,
           pool_vectors, W_key, q_proj, W_base, b_base, gamma,
           ln_scale, ln_bias, Wb1, bb1, Wb2, bb2, W_lm):
    b, t_tot = input_ids.shape
    vocab, d_a = embed_table.shape
    n_pool, d_pool = pool_vectors.shape

    ids = input_ids.reshape(t_tot).astype(jnp.int32)
    x = _embed_gather(embed_table, ids)
    pos = _pos_enc(t_tot, d_a)

    full = lambda i: (0, 0)
    ha, kq = pl.pallas_call(
        _parta_body,
        grid=(t_tot // _TBLK,),
        in_specs=[
            pl.BlockSpec((_TBLK, d_a), lambda i: (i, 0)),
            pl.BlockSpec((_TBLK, d_a), lambda i: (i, 0)),
            pl.BlockSpec((d_a, d_a), full),
            pl.BlockSpec((1, d_a), full),
            pl.BlockSpec((d_a, d_a), full),
            pl.BlockSpec((1, d_a), full),
            pl.BlockSpec((d_a, _D_K), full),
            pl.BlockSpec((d_pool, _D_K), full),
        ],
        out_specs=[
            pl.BlockSpec((_TBLK, d_a), lambda i: (i, 0)),
            pl.BlockSpec((1, d_pool), full),
        ],
        out_shape=[
            jax.ShapeDtypeStruct((t_tot, d_a), jnp.float32),
            jax.ShapeDtypeStruct((1, d_pool), jnp.float32),
        ],
        scratch_shapes=[pltpu.VMEM((1, d_a), jnp.float32)],
    )(x, pos.astype(jnp.float32), Wa1, ba1.reshape(1, -1), Wa2,
      ba2.reshape(1, -1), q_proj, W_key)

    lam = jnp.asarray(lambda_val, jnp.float32).reshape(1, 1)
    warm = jnp.where(is_warmup, 1.0, 0.0).astype(jnp.float32).reshape(1, 1)
    idxmat, alphamat = pl.pallas_call(
        _scan_body,
        grid=(n_pool // _PBLK,),
        in_specs=[
            pl.BlockSpec(memory_space=pltpu.SMEM),
            pl.BlockSpec(memory_space=pltpu.SMEM),
            pl.BlockSpec((1, d_pool), full),
            pl.BlockSpec((_PBLK, d_pool), lambda j: (j, 0)),
        ],
        out_specs=[
            pl.BlockSpec((_TOP_K, 128), full),
            pl.BlockSpec((_TOP_K, 128), full),
        ],
        out_shape=[
            jax.ShapeDtypeStruct((_TOP_K, 128), jnp.int32),
            jax.ShapeDtypeStruct((_TOP_K, 128), jnp.float32),
        ],
        scratch_shapes=[pltpu.VMEM((n_pool // _PBLK, _PBLK), jnp.float32)],
    )(lam, warm, kq, pool_vectors)

    idx8 = idxmat[:, 0].reshape(1, _TOP_K)
    gam = jnp.asarray(gamma, jnp.float32).reshape(1, 1)
    logits, w2d = pl.pallas_call(
        _final_body,
        grid=(vocab // _VBLK,),
        in_specs=[
            pl.BlockSpec(memory_space=pltpu.SMEM),
            pl.BlockSpec(memory_space=pltpu.SMEM),
            pl.BlockSpec((_TOP_K, 128), full),
            pl.BlockSpec((_D_B, d_a), full),
            pl.BlockSpec(memory_space=pl.ANY),
            pl.BlockSpec((t_tot, d_a), full),
            pl.BlockSpec((1, _D_B), full),
            pl.BlockSpec((1, _D_B), full),
            pl.BlockSpec((1, _D_B), full),
            pl.BlockSpec((_D_B, _D_B), full),
            pl.BlockSpec((1, _D_B), full),
            pl.BlockSpec((_D_B, _D_B), full),
            pl.BlockSpec((1, _D_B), full),
            pl.BlockSpec((_D_B, _VBLK), lambda v: (0, v)),
        ],
        out_specs=[
            pl.BlockSpec((t_tot, _VBLK), lambda v: (0, v)),
            pl.BlockSpec((_D_B, d_a), full),
        ],
        out_shape=[
            jax.ShapeDtypeStruct((t_tot, vocab), jnp.float32),
            jax.ShapeDtypeStruct((_D_B, d_a), jnp.float32),
        ],
        scratch_shapes=[
            pltpu.VMEM((_TOP_K, d_pool), jnp.float32),
            pltpu.SemaphoreType.DMA((_TOP_K,)),
            pltpu.VMEM((t_tot, _D_B), jnp.bfloat16),
        ],
    )(idx8, gam, alphamat, W_base, pool_vectors, ha, b_base.reshape(1, -1),
      ln_scale.reshape(1, -1), ln_bias.reshape(1, -1), Wb1,
      bb1.reshape(1, -1), Wb2, bb2.reshape(1, -1), W_lm)

    return (logits.reshape(b, t_tot, vocab),
            alphamat[:, 0].reshape(b, _TOP_K),
            idxmat[:, 0].reshape(b, _TOP_K),
            w2d.reshape(b, _D_B, d_a))
